# Initial kernel scaffold; baseline (speedup 1.0000x reference)
import jax
import jax.numpy as jnp
from jax.experimental import pallas as pl


def kernel(h, W1, b1, W2, b2, Wa, ba, Wfc, bfc):
    raise NotImplementedError("write your pallas kernel here")


# speedup vs baseline: 1.3185x; 1.3185x over previous
import functools

import jax
import jax.numpy as jnp
from jax.experimental import pallas as pl
from jax.experimental.pallas import tpu as pltpu

_B, _C, _N = 8, 128, 65536
_K = 2048
_H1, _H2 = 64, 64
_OUT = 144
_NT = 4096


def _imp_body(h_ref, W1_ref, b1_ref, W2_ref, b2_ref, Wa_ref, ba_ref, out_ref):
    hb = h_ref[0]
    f1 = jax.lax.dot(W1_ref[...], hb, preferred_element_type=jnp.float32)
    f1 = f1 + b1_ref[...].T
    f1 = jnp.where(f1 >= 0, f1, 0.1 * f1)
    f2 = jax.lax.dot(W2_ref[...], f1, preferred_element_type=jnp.float32)
    f2 = f2 + b2_ref[...].T
    f2 = jnp.where(f2 >= 0, f2, 0.1 * f2)
    a = jax.lax.dot(Wa_ref[...], f2, preferred_element_type=jnp.float32)
    a = a + ba_ref[...]
    out_ref[...] = jnp.clip(jax.nn.sigmoid(a), 0.0, 1.0)[None]


def _importance(h, W1, b1, W2, b2, Wa, ba):
    grid = (_B, _N // _NT)
    return pl.pallas_call(
        _imp_body,
        grid=grid,
        in_specs=[
            pl.BlockSpec((1, _C, _NT), lambda b, n: (b, 0, n)),
            pl.BlockSpec((_H1, _C), lambda b, n: (0, 0)),
            pl.BlockSpec((1, _H1), lambda b, n: (0, 0)),
            pl.BlockSpec((_H2, _H1), lambda b, n: (0, 0)),
            pl.BlockSpec((1, _H2), lambda b, n: (0, 0)),
            pl.BlockSpec((1, _H2), lambda b, n: (0, 0)),
            pl.BlockSpec((1, 1), lambda b, n: (0, 0)),
        ],
        out_specs=pl.BlockSpec((1, 1, _NT), lambda b, n: (b, 0, n)),
        out_shape=jax.ShapeDtypeStruct((_B, 1, _N), jnp.float32),
    )(h, W1, b1.reshape(1, _H1), W2, b2.reshape(1, _H2), Wa,
      ba.reshape(1, 1)).reshape(_B, _N)


def _fc_body(hs_ref, Wfc_ref, bfc_ref, out_ref):
    hs = hs_ref[0]
    y = jax.lax.dot_general(
        hs, Wfc_ref[...],
        dimension_numbers=(((0,), (1,)), ((), ())),
        preferred_element_type=jnp.float32,
    )
    out_ref[...] = (y + bfc_ref[...])[None]


def _fc(h_sub, Wfc, bfc):
    return pl.pallas_call(
        _fc_body,
        grid=(_B,),
        in_specs=[
            pl.BlockSpec((1, _C, _K), lambda b: (b, 0, 0)),
            pl.BlockSpec((_OUT, _C), lambda b: (0, 0)),
            pl.BlockSpec((1, _OUT), lambda b: (0, 0)),
        ],
        out_specs=pl.BlockSpec((1, _K, _OUT), lambda b: (b, 0, 0)),
        out_shape=jax.ShapeDtypeStruct((_B, _K, _OUT), jnp.float32),
    )(h_sub, Wfc, bfc.reshape(1, _OUT))


def kernel(h, W1, b1, W2, b2, Wa, ba, Wfc, bfc):
    importance = _importance(h, W1, b1, W2, b2, Wa, ba)
    _, inds = jax.lax.top_k(importance, _K)
    inds = jnp.clip(inds, 0, _N - 1)
    h_sub = jnp.take_along_axis(h, inds[:, None, :], axis=2)
    x = _fc(h_sub, Wfc, bfc)
    x = x.reshape(_B, _K, 3, 6, 8)
    return (x, inds, importance)


# speedup vs baseline: 1.3321x; 1.0103x over previous
import functools

import jax
import jax.numpy as jnp
from jax.experimental import pallas as pl
from jax.experimental.pallas import tpu as pltpu

_B, _C, _N = 8, 128, 65536
_K = 2048
_H1, _H2 = 64, 64
_OUT = 144
_NT = 4096


def _imp_body(h_ref, W1_ref, b1_ref, W2_ref, b2_ref, Wa_ref, ba_ref, out_ref):
    hb = h_ref[0]
    f1 = jax.lax.dot(W1_ref[...], hb, preferred_element_type=jnp.float32)
    f1 = f1 + b1_ref[...].T
    f1 = jnp.where(f1 >= 0, f1, 0.1 * f1)
    f2 = jax.lax.dot(W2_ref[...], f1, preferred_element_type=jnp.float32)
    f2 = f2 + b2_ref[...].T
    f2 = jnp.where(f2 >= 0, f2, 0.1 * f2)
    a = jax.lax.dot(Wa_ref[...], f2, preferred_element_type=jnp.float32)
    a = a + ba_ref[...]
    out_ref[...] = jnp.clip(jax.nn.sigmoid(a), 0.0, 1.0)[None]


def _importance(h, W1, b1, W2, b2, Wa, ba):
    grid = (_B, _N // _NT)
    return pl.pallas_call(
        _imp_body,
        grid=grid,
        in_specs=[
            pl.BlockSpec((1, _C, _NT), lambda b, n: (b, 0, n)),
            pl.BlockSpec((_H1, _C), lambda b, n: (0, 0)),
            pl.BlockSpec((1, _H1), lambda b, n: (0, 0)),
            pl.BlockSpec((_H2, _H1), lambda b, n: (0, 0)),
            pl.BlockSpec((1, _H2), lambda b, n: (0, 0)),
            pl.BlockSpec((1, _H2), lambda b, n: (0, 0)),
            pl.BlockSpec((1, 1), lambda b, n: (0, 0)),
        ],
        out_specs=pl.BlockSpec((1, 1, _NT), lambda b, n: (b, 0, n)),
        out_shape=jax.ShapeDtypeStruct((_B, 1, _N), jnp.float32),
    )(h, W1, b1.reshape(1, _H1), W2, b2.reshape(1, _H2), Wa,
      ba.reshape(1, 1)).reshape(_B, _N)


def _fc_body(hs_ref, Wfc_ref, bfc_ref, out_ref):
    hs = hs_ref[0]
    y = jax.lax.dot_general(
        hs, Wfc_ref[...],
        dimension_numbers=(((0,), (1,)), ((), ())),
        preferred_element_type=jnp.float32,
    )
    out_ref[...] = (y + bfc_ref[...])[None]


def _fc(h_sub, Wfc, bfc):
    return pl.pallas_call(
        _fc_body,
        grid=(_B,),
        in_specs=[
            pl.BlockSpec((1, _C, _K), lambda b: (b, 0, 0)),
            pl.BlockSpec((_OUT, _C), lambda b: (0, 0)),
            pl.BlockSpec((1, _OUT), lambda b: (0, 0)),
        ],
        out_specs=pl.BlockSpec((1, _K, _OUT), lambda b: (b, 0, 0)),
        out_shape=jax.ShapeDtypeStruct((_B, _K, _OUT), jnp.float32),
    )(h_sub, Wfc, bfc.reshape(1, _OUT))


def kernel(h, W1, b1, W2, b2, Wa, ba, Wfc, bfc):
    importance = _importance(h, W1, b1, W2, b2, Wa, ba)
    _, inds = jax.lax.top_k(importance, _K)
    inds = jnp.clip(inds, 0, _N - 1)
    h_sub = jnp.take_along_axis(h, inds[:, None, :], axis=2,
                                mode="clip")
    x = _fc(h_sub, Wfc, bfc)
    x = x.reshape(_B, _K, 3, 6, 8)
    return (x, inds, importance)


# speedup vs baseline: 1.6536x; 1.2414x over previous
import functools

import jax
import jax.numpy as jnp
from jax import lax
from jax.experimental import pallas as pl
from jax.experimental.pallas import tpu as pltpu
from jax.experimental.pallas import tpu_sc as plsc

_B, _C, _N = 8, 128, 65536
_K = 2048
_H1, _H2 = 64, 64
_OUT = 144
_NT = 4096
_L = 16
_NBINS = 1024
_CAND = 4096
_PAD_KEY = 0x3FFFFFFF


def _imp_body(h_ref, W1_ref, b1_ref, W2_ref, b2_ref, Wa_ref, ba_ref, out_ref):
    hb = h_ref[0]
    f1 = jax.lax.dot(W1_ref[...], hb, preferred_element_type=jnp.float32)
    f1 = f1 + b1_ref[...].T
    f1 = jnp.where(f1 >= 0, f1, 0.1 * f1)
    f2 = jax.lax.dot(W2_ref[...], f1, preferred_element_type=jnp.float32)
    f2 = f2 + b2_ref[...].T
    f2 = jnp.where(f2 >= 0, f2, 0.1 * f2)
    a = jax.lax.dot(Wa_ref[...], f2, preferred_element_type=jnp.float32)
    a = a + ba_ref[...]
    out_ref[...] = jnp.clip(jax.nn.sigmoid(a), 0.0, 1.0)[None]


def _importance(h, W1, b1, W2, b2, Wa, ba):
    grid = (_B, _N // _NT)
    return pl.pallas_call(
        _imp_body,
        grid=grid,
        in_specs=[
            pl.BlockSpec((1, _C, _NT), lambda b, n: (b, 0, n)),
            pl.BlockSpec((_H1, _C), lambda b, n: (0, 0)),
            pl.BlockSpec((1, _H1), lambda b, n: (0, 0)),
            pl.BlockSpec((_H2, _H1), lambda b, n: (0, 0)),
            pl.BlockSpec((1, _H2), lambda b, n: (0, 0)),
            pl.BlockSpec((1, _H2), lambda b, n: (0, 0)),
            pl.BlockSpec((1, 1), lambda b, n: (0, 0)),
        ],
        out_specs=pl.BlockSpec((1, 1, _NT), lambda b, n: (b, 0, n)),
        out_shape=jax.ShapeDtypeStruct((_B, 1, _N), jnp.float32),
    )(h, W1, b1.reshape(1, _H1), W2, b2.reshape(1, _H2), Wa,
      ba.reshape(1, 1)).reshape(_B, _N)


_U = 4


def _sc_topk_body(imp_hbm, inds_hbm, row_v, hist_v, cka, cia, ckb, cib):
    cid = lax.axis_index("c")
    sid = lax.axis_index("s")
    row = cid * (_B // 2) + sid

    @pl.when(sid < _B // 2)
    def _():
        pltpu.sync_copy(imp_hbm.at[row], row_v)
        zeros = jnp.zeros((_L,), jnp.int32)

        def hist_step(g, _):
            for u in range(_U):
                i = g * _U + u
                bits = lax.bitcast_convert_type(row_v[pl.ds(i * _L, _L)],
                                                jnp.int32)
                b = jnp.minimum(bits >> 20, _NBINS - 1)
                cnt, last = plsc.scan_count(b)
                plsc.addupdate_scatter(hist_v, [b], cnt, mask=last)
            return 0

        def zero_step(i, _):
            hist_v[pl.ds(i * _L, _L)] = zeros
            return 0

        lax.fori_loop(0, _NBINS // _L, zero_step, 0)
        lax.fori_loop(0, _N // _L // _U, hist_step, 0)

        def bin_step(j, carry):
            above, found, bstar = carry
            i = _NBINS // _L - 1 - j
            hvec = hist_v[pl.ds(i * _L, _L)]
            sfx = lax.rev(plsc.cumsum(lax.rev(hvec, (0,))), (0,))
            incl = sfx + above
            npass = jnp.sum((incl >= _K).astype(jnp.int32))
            cand = i * _L + npass - 1
            bstar = jnp.where(jnp.logical_and(jnp.logical_not(found),
                                              npass > 0), cand, bstar)
            found = jnp.logical_or(found, npass > 0)
            return above + jnp.sum(hvec), found, bstar

        _, _, bstar = lax.fori_loop(
            0, _NBINS // _L, bin_step,
            (jnp.int32(0), jnp.bool_(False), jnp.int32(0)))
        thresh = bstar << 20

        def pad_step(i, _):
            cka[pl.ds(i * _L, _L)] = jnp.full((_L,), _PAD_KEY, jnp.int32)
            return 0

        lax.fori_loop(0, _CAND // _L, pad_step, 0)
        lane = lax.iota(jnp.int32, _L)

        def compact_step(g, off):
            def do_store(off):
                for u in range(_U):
                    i = g * _U + u
                    bits = lax.bitcast_convert_type(
                        row_v[pl.ds(i * _L, _L)], jnp.int32)
                    m = bits >= thresh
                    plsc.store_compressed(cka.at[pl.ds(off, _L)],
                                          _PAD_KEY - bits, mask=m)
                    plsc.store_compressed(cia.at[pl.ds(off, _L)],
                                          i * _L + lane, mask=m)
                    off = off + jnp.sum(m.astype(jnp.int32))
                return off

            return lax.cond(off <= _CAND - _U * _L, do_store,
                            lambda o: o, off)

        ncand = lax.fori_loop(0, _N // _L // _U, compact_step, jnp.int32(0))
        ngrp = (ncand + _U * _L - 1) // (_U * _L)

        for shift, (sk, si, dk, di) in zip(
                (0, 10, 20), ((cka, cia, ckb, cib),
                              (ckb, cib, cka, cia),
                              (cka, cia, ckb, cib))):
            lax.fori_loop(0, _NBINS // _L, zero_step, 0)

            def dig_hist(g, _, sk=sk, shift=shift):
                for u in range(_U):
                    i = g * _U + u
                    d = (sk[pl.ds(i * _L, _L)] >> shift) & (_NBINS - 1)
                    cnt, last = plsc.scan_count(d)
                    plsc.addupdate_scatter(hist_v, [d], cnt, mask=last)
                return 0

            lax.fori_loop(0, ngrp, dig_hist, 0)

            def excl_step(i, carry):
                hvec = hist_v[pl.ds(i * _L, _L)]
                hist_v[pl.ds(i * _L, _L)] = (plsc.cumsum(hvec) - hvec
                                             + carry)
                return carry + jnp.sum(hvec)

            lax.fori_loop(0, _NBINS // _L, excl_step, jnp.int32(0))

            def permute(g, _, sk=sk, si=si, dk=dk, di=di, shift=shift):
                for u in range(_U):
                    i = g * _U + u
                    k = sk[pl.ds(i * _L, _L)]
                    v = si[pl.ds(i * _L, _L)]
                    d = (k >> shift) & (_NBINS - 1)
                    cnt, last = plsc.scan_count(d)
                    pos = plsc.load_gather(hist_v, [d]) + cnt - 1
                    plsc.store_scatter(dk, [pos], k)
                    plsc.store_scatter(di, [pos], v)
                    plsc.addupdate_scatter(hist_v, [d], cnt, mask=last)
                return 0

            lax.fori_loop(0, ngrp, permute, 0)

        pltpu.sync_copy(cib.at[pl.ds(0, _K)], inds_hbm.at[row])


def _sc_topk(imp):
    mesh = plsc.VectorSubcoreMesh(core_axis_name="c", subcore_axis_name="s",
                                  num_cores=2, num_subcores=16)
    return pl.kernel(
        _sc_topk_body,
        out_type=jax.ShapeDtypeStruct((_B, _K), jnp.int32),
        mesh=mesh,
        compiler_params=pltpu.CompilerParams(needs_layout_passes=False),
        scratch_types=[
            pltpu.VMEM((_N,), jnp.float32),
            pltpu.VMEM((_NBINS,), jnp.int32),
            pltpu.VMEM((_CAND,), jnp.int32),
            pltpu.VMEM((_CAND,), jnp.int32),
            pltpu.VMEM((_CAND,), jnp.int32),
            pltpu.VMEM((_CAND,), jnp.int32),
        ],
    )(imp)


def _fc_body(hs_ref, Wfc_ref, bfc_ref, out_ref):
    hs = hs_ref[0]
    y = jax.lax.dot_general(
        hs, Wfc_ref[...],
        dimension_numbers=(((0,), (1,)), ((), ())),
        preferred_element_type=jnp.float32,
    )
    out_ref[...] = (y + bfc_ref[...])[None]


def _fc(h_sub, Wfc, bfc):
    return pl.pallas_call(
        _fc_body,
        grid=(_B,),
        in_specs=[
            pl.BlockSpec((1, _C, _K), lambda b: (b, 0, 0)),
            pl.BlockSpec((_OUT, _C), lambda b: (0, 0)),
            pl.BlockSpec((1, _OUT), lambda b: (0, 0)),
        ],
        out_specs=pl.BlockSpec((1, _K, _OUT), lambda b: (b, 0, 0)),
        out_shape=jax.ShapeDtypeStruct((_B, _K, _OUT), jnp.float32),
    )(h_sub, Wfc, bfc.reshape(1, _OUT))


def kernel(h, W1, b1, W2, b2, Wa, ba, Wfc, bfc):
    importance = _importance(h, W1, b1, W2, b2, Wa, ba)
    inds = _sc_topk(importance)
    h_sub = jnp.take_along_axis(h, inds[:, None, :], axis=2,
                                mode="clip")
    x = _fc(h_sub, Wfc, bfc)
    x = x.reshape(_B, _K, 3, 6, 8)
    return (x, inds, importance)
